# baseline (speedup 1.0000x reference)
import jax
import jax.numpy as jnp
from jax import lax
from jax.experimental import pallas as pl
from jax.experimental.pallas import tpu as pltpu
from jax.experimental.pallas import tpu_sc as plsc

_N, _F, _H0, _H1, _K, _V = 10000, 128, 64, 32, 16, 1
_BM = 400
_bf16, _f32 = jnp.bfloat16, jnp.float32

_NC, _NS = 2, 16
_NW = _NC * _NS
_NP = 10240
_RPW = _NP // _NW
_GROUPS = _RPW // 16


def _decoder_body(e3_ref, et_ref, w1d_ref, b1d_ref, w2d_ref, b2d_ref,
                  recon_ref, xbar_ref):
    lhs = e3_ref[0]
    recon_ref[...] = lax.dot_general(
        lhs, et_ref[...], (((0,), (0,)), ((), ())),
        preferred_element_type=_f32)
    e = lhs.T
    mid = lax.dot_general(e, w1d_ref[...], (((1,), (0,)), ((), ())),
                          preferred_element_type=_f32) + b1d_ref[...]
    mid_bf = jnp.maximum(mid, 0.0).astype(_bf16)
    xbar_ref[...] = lax.dot_general(
        mid_bf, w2d_ref[...], (((1,), (0,)), ((), ())),
        preferred_element_type=_f32) + b2d_ref[...]


def _bfly_min(x, iota):
    for k in (1, 2, 4, 8):
        x = jnp.minimum(x, x.at[iota ^ k].get(mode="promise_in_bounds"))
    return x


def _bfly_sum(x, iota):
    for k in (1, 2, 4, 8):
        x = x + x.at[iota ^ k].get(mode="promise_in_bounds")
    return x


def _assign_sc(dist_hbm, q_hbm, t1_hbm, t2_hbm, dist_v, q_v, t1_v, t2_v):
    wid = lax.axis_index("s") * _NC + lax.axis_index("c")
    base = wid * _RPW
    pltpu.sync_copy(dist_hbm.at[pl.ds(base, _RPW)], dist_v)
    iota = lax.iota(jnp.int32, 16)

    def group(g, carry):
        t1acc = jnp.zeros((16,), jnp.int32)
        t2acc = jnp.zeros((16,), jnp.int32)
        for r in range(16):
            row = dist_v[g * 16 + r, :]
            qr = 1.0 / (1.0 + row)
            q_v[g * 16 + r, :] = qr / _bfly_sum(qr, iota)
            m1 = _bfly_min(row, iota)
            t1 = _bfly_min(jnp.where(row == m1, iota, 16), iota)
            masked = jnp.where(iota == t1, jnp.inf, row)
            m2 = _bfly_min(masked, iota)
            t2 = _bfly_min(jnp.where(masked == m2, iota, 16), iota)
            t1acc = jnp.where(iota == r, t1, t1acc)
            t2acc = jnp.where(iota == r, t2, t2acc)
        t1_v[pl.ds(g * 16, 16)] = t1acc
        t2_v[pl.ds(g * 16, 16)] = t2acc
        return carry

    lax.fori_loop(0, _GROUPS, group, 0)
    pltpu.sync_copy(q_v, q_hbm.at[pl.ds(base, _RPW)])
    pltpu.sync_copy(t1_v, t1_hbm.at[pl.ds(base, _RPW)])
    pltpu.sync_copy(t2_v, t2_hbm.at[pl.ds(base, _RPW)])


def kernel(features, adjs, input_view, W_s1, W_s2, dec_W1, dec_b1, dec_W2,
           dec_b2, cluster_layer):
    adj = adjs.reshape(_N, _N)

    xw1_bf = lax.dot_general(features, W_s1, (((1,), (0,)), ((), ())),
                             preferred_element_type=_bf16)
    h1preT = lax.dot_general(xw1_bf, adj, (((0,), (1,)), ((), ())),
                             preferred_element_type=_bf16)
    h1T_bf = jnp.maximum(h1preT.astype(_f32), 0.0).astype(_bf16)
    h1w2_bf = lax.dot_general(h1T_bf, W_s2, (((0,), (0,)), ((), ())),
                              preferred_element_type=_bf16)
    embT = jnp.maximum(lax.dot_general(h1w2_bf, adj, (((0,), (1,)), ((), ())),
                                       preferred_element_type=_f32), 0.0)
    emb = embT.T

    diff = emb[:, None, :] - cluster_layer[None, :, :]
    dist = jnp.sum(diff * diff, axis=2)

    emb3d = jnp.moveaxis(embT.reshape(_H1, _N // _BM, _BM), 1, 0)
    recon2d, x_bar = pl.pallas_call(
        _decoder_body,
        grid=(_N // _BM,),
        in_specs=[
            pl.BlockSpec((1, _H1, _BM), lambda i: (i, 0, 0)),
            pl.BlockSpec((_H1, _N), lambda i: (0, 0)),
            pl.BlockSpec((_H1, _H0), lambda i: (0, 0)),
            pl.BlockSpec((1, _H0), lambda i: (0, 0)),
            pl.BlockSpec((_H0, _F), lambda i: (0, 0)),
            pl.BlockSpec((1, _F), lambda i: (0, 0)),
        ],
        out_specs=[
            pl.BlockSpec((_BM, _N), lambda i: (i, 0)),
            pl.BlockSpec((_BM, _F), lambda i: (i, 0)),
        ],
        out_shape=[
            jax.ShapeDtypeStruct((_N, _N), _f32),
            jax.ShapeDtypeStruct((_N, _F), _f32),
        ],
        compiler_params=pltpu.CompilerParams(
            dimension_semantics=("arbitrary",)),
    )(emb3d, embT, dec_W1, dec_b1.reshape(1, _H0), dec_W2,
      dec_b2.reshape(1, _F))
    recon = recon2d[None]

    dist_pad = jnp.pad(dist, ((0, _NP - _N), (0, 0)))
    q_pad, t1_pad, t2_pad = pl.kernel(
        _assign_sc,
        out_type=[
            jax.ShapeDtypeStruct((_NP, _K), _f32),
            jax.ShapeDtypeStruct((_NP,), jnp.int32),
            jax.ShapeDtypeStruct((_NP,), jnp.int32),
        ],
        mesh=plsc.VectorSubcoreMesh(core_axis_name="c", subcore_axis_name="s"),
        scratch_types=[
            pltpu.VMEM((_RPW, _K), _f32),
            pltpu.VMEM((_RPW, _K), _f32),
            pltpu.VMEM((_RPW,), jnp.int32),
            pltpu.VMEM((_RPW,), jnp.int32),
        ],
    )(dist_pad)
    q = q_pad[:_N]
    top1 = t1_pad[:_N]
    top2 = t2_pad[:_N]

    return (emb, recon, x_bar, q, top1, top2, cluster_layer)
